# baseline (speedup 1.0000x reference)
import functools

import jax
import jax.numpy as jnp
from jax import lax
from jax.experimental import pallas as pl
from jax.experimental.pallas import tpu as pltpu
from jax.experimental.pallas import tpu_sc as plsc

B, L, V, D, O = 4096, 50, 1000000, 64, 1

NC, NS = 2, 16
NW = NC * NS
BPW = B // NW
IPW = BPW * L
NG = BPW // 16

MV_BLK = 8192


def _mv_body(t_ref, w_ref, o_ref):
    o_ref[...] = jnp.dot(t_ref[...], w_ref[...],
                         preferred_element_type=jnp.float32)[:, 0]


def _table_matvec(table, w2d):
    return pl.pallas_call(
        _mv_body,
        grid=(pl.cdiv(V, MV_BLK),),
        in_specs=[
            pl.BlockSpec((MV_BLK, D), lambda i: (i, 0)),
            pl.BlockSpec((D, 1), lambda i: (0, 0)),
        ],
        out_specs=pl.BlockSpec((MV_BLK,), lambda i: (i,)),
        out_shape=jax.ShapeDtypeStruct((V,), jnp.float32),
    )(table, w2d)


_mesh = plsc.VectorSubcoreMesh(core_axis_name="c", subcore_axis_name="s")


@functools.partial(
    pl.kernel,
    out_type=jax.ShapeDtypeStruct((B,), jnp.float32),
    mesh=_mesh,
    scratch_types=[
        pltpu.VMEM((IPW,), jnp.int32),
        pltpu.VMEM((BPW,), jnp.int32),
        pltpu.VMEM((16,), jnp.float32),
        pltpu.VMEM((IPW,), jnp.float32),
        pltpu.VMEM((BPW,), jnp.float32),
        pltpu.SemaphoreType.DMA,
    ],
    compiler_params=pltpu.CompilerParams(
        needs_layout_passes=False, use_tc_tiling_on_sc=False),
)
def _sc_pool(t_h, xf_h, lens_h, b_h, out_h,
             x_v, lens_v, b_v, tv_v, out_v, sem):
    wid = lax.axis_index("s") * NC + lax.axis_index("c")
    bbase = wid * BPW

    pltpu.sync_copy(xf_h.at[pl.ds(bbase * L, IPW)], x_v)
    pltpu.sync_copy(lens_h.at[pl.ds(bbase, BPW)], lens_v)
    pltpu.sync_copy(b_h, b_v)
    pltpu.async_copy(t_h.at[x_v], tv_v, sem).wait()

    bvec = b_v[...]
    zero = jnp.zeros((16,), jnp.float32)
    lane = lax.iota(jnp.int32, 16)

    for g in range(NG):
        len_vec = lens_v[pl.ds(g * 16, 16)]
        base = (g * 16 + lane) * L
        acc = zero
        for j in range(L):
            v = plsc.load_gather(tv_v, [base + j])
            acc = acc + jnp.where(len_vec > j, v, zero)
        out_v[pl.ds(g * 16, 16)] = acc / len_vec.astype(jnp.float32) + bvec

    pltpu.sync_copy(out_v, out_h.at[pl.ds(bbase, BPW)])


def kernel(x, lens, table, W, b):
    xf = x.astype(jnp.int32).reshape(B * L)
    lens32 = lens.astype(jnp.int32)
    w2d = W.reshape(O, D).astype(jnp.float32).T
    bv = jnp.broadcast_to(b.astype(jnp.float32), (16,))
    t = _table_matvec(table.astype(jnp.float32), w2d)
    out = _sc_pool(t, xf, lens32, bv)
    return out.reshape(B, O)


# speedup vs baseline: 1.4090x; 1.4090x over previous
import functools

import jax
import jax.numpy as jnp
from jax import lax
from jax.experimental import pallas as pl
from jax.experimental.pallas import tpu as pltpu
from jax.experimental.pallas import tpu_sc as plsc

B, L, V, D, O = 4096, 50, 1000000, 64, 1

NC, NS = 2, 16
NW = NC * NS
BPW = B // NW
IPW = BPW * L
NG = BPW // 16

MV_BLK = 8192


def _mv_body(t_ref, wb_ref, m_ref, o_ref):
    c1 = jnp.dot(t_ref[...], wb_ref[...], preferred_element_type=jnp.float32)
    c3 = c1.reshape(MV_BLK // D, D, D) * m_ref[...][None, :, :]
    o_ref[...] = jnp.sum(c3, axis=1)


def _table_matvec(table, wb, mask):
    return pl.pallas_call(
        _mv_body,
        grid=(pl.cdiv(V, MV_BLK),),
        in_specs=[
            pl.BlockSpec((MV_BLK, D), lambda i: (i, 0)),
            pl.BlockSpec((D, D), lambda i: (0, 0)),
            pl.BlockSpec((D, D), lambda i: (0, 0)),
        ],
        out_specs=pl.BlockSpec((MV_BLK // D, D), lambda i: (i, 0)),
        out_shape=jax.ShapeDtypeStruct((V // D, D), jnp.float32),
    )(table, wb, mask)


_mesh = plsc.VectorSubcoreMesh(core_axis_name="c", subcore_axis_name="s")


@functools.partial(
    pl.kernel,
    out_type=jax.ShapeDtypeStruct((B,), jnp.float32),
    mesh=_mesh,
    scratch_types=[
        pltpu.VMEM((IPW,), jnp.int32),
        pltpu.VMEM((BPW,), jnp.int32),
        pltpu.VMEM((16,), jnp.float32),
        pltpu.VMEM((IPW,), jnp.float32),
        pltpu.VMEM((BPW,), jnp.float32),
        pltpu.SemaphoreType.DMA,
    ],
    compiler_params=pltpu.CompilerParams(
        needs_layout_passes=False, use_tc_tiling_on_sc=False),
)
def _sc_pool(t_h, xf_h, lens_h, b_h, out_h,
             x_v, lens_v, b_v, tv_v, out_v, sem):
    wid = lax.axis_index("s") * NC + lax.axis_index("c")
    bbase = wid * BPW

    pltpu.sync_copy(xf_h.at[pl.ds(bbase * L, IPW)], x_v)
    pltpu.sync_copy(lens_h.at[pl.ds(bbase, BPW)], lens_v)
    pltpu.sync_copy(b_h, b_v)
    pltpu.async_copy(t_h.at[x_v], tv_v, sem).wait()

    bvec = b_v[...]
    zero = jnp.zeros((16,), jnp.float32)
    lane = lax.iota(jnp.int32, 16)

    for g in range(NG):
        len_vec = lens_v[pl.ds(g * 16, 16)]
        base = (g * 16 + lane) * L
        acc = zero
        for j in range(L):
            v = plsc.load_gather(tv_v, [base + j])
            acc = acc + jnp.where(len_vec > j, v, zero)
        out_v[pl.ds(g * 16, 16)] = acc / len_vec.astype(jnp.float32) + bvec

    pltpu.sync_copy(out_v, out_h.at[pl.ds(bbase, BPW)])


def kernel(x, lens, table, W, b):
    xf = x.astype(jnp.int32).reshape(B * L)
    lens32 = lens.astype(jnp.int32)
    wv = W.reshape(D).astype(jnp.float32)
    wb = jnp.broadcast_to(wv[:, None], (D, D))
    mask = jnp.eye(D, dtype=jnp.float32)
    bv = jnp.broadcast_to(b.astype(jnp.float32), (16,))
    t = _table_matvec(table.astype(jnp.float32), wb, mask).reshape(V)
    out = _sc_pool(t, xf, lens32, bv)
    return out.reshape(B, O)


# speedup vs baseline: 1.4926x; 1.0594x over previous
import functools

import jax
import jax.numpy as jnp
from jax import lax
from jax.experimental import pallas as pl
from jax.experimental.pallas import tpu as pltpu
from jax.experimental.pallas import tpu_sc as plsc

B, L, V, D, O = 4096, 50, 1000000, 64, 1

NC, NS = 2, 16
NW = NC * NS
BPW = B // NW
IPW = BPW * L
NG = BPW // 16

MV_BLK = 32768


def _mv_body(t_ref, wb_ref, m_ref, o_ref):
    c1 = jnp.dot(t_ref[...], wb_ref[...], preferred_element_type=jnp.float32)
    c3 = c1.reshape(MV_BLK // D, D, D) * m_ref[...][None, :, :]
    o_ref[...] = jnp.sum(c3, axis=1)


def _table_matvec(table, wb, mask):
    return pl.pallas_call(
        _mv_body,
        grid=(pl.cdiv(V, MV_BLK),),
        in_specs=[
            pl.BlockSpec((MV_BLK, D), lambda i: (i, 0)),
            pl.BlockSpec((D, D), lambda i: (0, 0)),
            pl.BlockSpec((D, D), lambda i: (0, 0)),
        ],
        out_specs=pl.BlockSpec((MV_BLK // D, D), lambda i: (i, 0)),
        out_shape=jax.ShapeDtypeStruct((V // D, D), jnp.float32),
    )(table, wb, mask)


_mesh = plsc.VectorSubcoreMesh(core_axis_name="c", subcore_axis_name="s")


@functools.partial(
    pl.kernel,
    out_type=jax.ShapeDtypeStruct((B,), jnp.float32),
    mesh=_mesh,
    scratch_types=[
        pltpu.VMEM((IPW,), jnp.int32),
        pltpu.VMEM((BPW,), jnp.int32),
        pltpu.VMEM((16,), jnp.float32),
        pltpu.VMEM((IPW,), jnp.float32),
        pltpu.VMEM((BPW,), jnp.float32),
        pltpu.SemaphoreType.DMA,
    ],
    compiler_params=pltpu.CompilerParams(
        needs_layout_passes=False, use_tc_tiling_on_sc=False),
)
def _sc_pool(t_h, xf_h, lens_h, b_h, out_h,
             x_v, lens_v, b_v, tv_v, out_v, sem):
    wid = lax.axis_index("s") * NC + lax.axis_index("c")
    bbase = wid * BPW

    pltpu.sync_copy(xf_h.at[pl.ds(bbase * L, IPW)], x_v)
    pltpu.sync_copy(lens_h.at[pl.ds(bbase, BPW)], lens_v)
    pltpu.sync_copy(b_h, b_v)
    pltpu.async_copy(t_h.at[x_v], tv_v, sem).wait()

    bvec = b_v[...]
    zero = jnp.zeros((16,), jnp.float32)
    lane = lax.iota(jnp.int32, 16)

    for g in range(NG):
        len_vec = lens_v[pl.ds(g * 16, 16)]
        base = (g * 16 + lane) * L
        acc = zero
        for j in range(L):
            v = plsc.load_gather(tv_v, [base + j])
            acc = acc + jnp.where(len_vec > j, v, zero)
        out_v[pl.ds(g * 16, 16)] = acc / len_vec.astype(jnp.float32) + bvec

    pltpu.sync_copy(out_v, out_h.at[pl.ds(bbase, BPW)])


def kernel(x, lens, table, W, b):
    xf = x.astype(jnp.int32).reshape(B * L)
    lens32 = lens.astype(jnp.int32)
    wv = W.reshape(D).astype(jnp.float32)
    wb = jnp.broadcast_to(wv[:, None], (D, D))
    mask = jnp.eye(D, dtype=jnp.float32)
    bv = jnp.broadcast_to(b.astype(jnp.float32), (16,))
    t = _table_matvec(table.astype(jnp.float32), wb, mask).reshape(V)
    out = _sc_pool(t, xf, lens32, bv)
    return out.reshape(B, O)
